# Initial kernel scaffold; baseline (speedup 1.0000x reference)
import jax
import jax.numpy as jnp
from jax.experimental import pallas as pl


def kernel(x, edge_index, W, b):
    raise NotImplementedError("write your pallas kernel here")


# speedup vs baseline: 23.9149x; 23.9149x over previous
import functools

import jax
import jax.numpy as jnp
from jax import lax
from jax.experimental import pallas as pl
from jax.experimental.pallas import tpu as pltpu
from jax.experimental.pallas import tpu_sc as plsc

N = 10000
E = 320000
D_IN = 128
D_OUT = 16

NC = 2
NS = 16
NW = NC * NS
CH = 128
NCHUNK = 80
E_PAD = NW * NCHUNK * CH
NP = 10240
RPT = NP // NS

_mesh = plsc.VectorSubcoreMesh(core_axis_name="c", subcore_axis_name="s")
_sc_params = pltpu.CompilerParams(use_tc_tiling_on_sc=False)


@functools.partial(
    pl.kernel,
    mesh=_mesh,
    out_type=jax.ShapeDtypeStruct((NC, NP, D_OUT), jnp.float32),
    scratch_types=[
        pltpu.VMEM((CH,), jnp.int32),
        pltpu.VMEM((CH, D_OUT), jnp.float32),
        pltpu.VMEM_SHARED((NP, D_OUT), jnp.float32),
    ],
    compiler_params=_sc_params,
)
def _deg_pass(col_hbm, ones_hbm, zeros_hbm, deg_hbm, idx_v, one_v, deg_sh):
    c = lax.axis_index("c")
    s = lax.axis_index("s")
    wid = c * NS + s
    pltpu.sync_copy(ones_hbm, one_v)
    pltpu.sync_copy(zeros_hbm, deg_sh.at[pl.ds(s * RPT, RPT)])
    plsc.subcore_barrier()

    def body(j, carry):
        pltpu.sync_copy(col_hbm.at[wid, j], idx_v)
        pltpu.sync_copy(one_v, deg_sh.at[idx_v], add=True)
        return carry

    lax.fori_loop(0, NCHUNK, body, 0)
    plsc.subcore_barrier()
    pltpu.sync_copy(
        deg_sh.at[pl.ds(s * RPT, RPT)], deg_hbm.at[c, pl.ds(s * RPT, RPT)]
    )


@functools.partial(
    pl.kernel,
    mesh=_mesh,
    out_type=jax.ShapeDtypeStruct((NC, NP, D_OUT), jnp.float32),
    scratch_types=[
        pltpu.VMEM((CH,), jnp.int32),
        pltpu.VMEM((CH,), jnp.int32),
        pltpu.VMEM((CH, D_OUT), jnp.float32),
        pltpu.VMEM_SHARED((NP, D_OUT), jnp.float32),
        pltpu.SemaphoreType.DMA,
    ],
    compiler_params=_sc_params,
)
def _edge_pass(y_hbm, row_hbm, col_hbm, zeros_hbm, acc_hbm,
               rid_v, cid_v, rows_v, acc_sh, sem):
    c = lax.axis_index("c")
    s = lax.axis_index("s")
    wid = c * NS + s
    pltpu.sync_copy(zeros_hbm, acc_sh.at[pl.ds(s * RPT, RPT)])
    plsc.subcore_barrier()

    def body(j, carry):
        pltpu.sync_copy(row_hbm.at[wid, j], rid_v)
        pltpu.sync_copy(col_hbm.at[wid, j], cid_v)
        pltpu.async_copy(y_hbm.at[rid_v], rows_v, sem).wait()
        pltpu.sync_copy(rows_v, acc_sh.at[cid_v], add=True)
        return carry

    lax.fori_loop(0, NCHUNK, body, 0)
    plsc.subcore_barrier()
    pltpu.sync_copy(
        acc_sh.at[pl.ds(s * RPT, RPT)], acc_hbm.at[c, pl.ds(s * RPT, RPT)]
    )


def _xw_body(x_ref, w_ref, d0_ref, d1_ref, y_ref, dis_ref):
    deg = d0_ref[...] + d1_ref[...] + 1.0
    dis = lax.rsqrt(deg)
    xw = jnp.dot(x_ref[...], w_ref[...], preferred_element_type=jnp.float32)
    y_ref[...] = xw * dis
    dis_ref[...] = dis


def _xw_call(x, W, d0, d1):
    return pl.pallas_call(
        _xw_body,
        out_shape=[
            jax.ShapeDtypeStruct((N, D_OUT), jnp.float32),
            jax.ShapeDtypeStruct((N, D_OUT), jnp.float32),
        ],
    )(x, W, d0, d1)


def _fin_body(a0_ref, a1_ref, y_ref, dis_ref, b_ref, out_ref):
    t = (a0_ref[...] + a1_ref[...] + y_ref[...]) * dis_ref[...] + b_ref[...]
    m = jnp.max(t, axis=1, keepdims=True)
    ls = jnp.log(jnp.sum(jnp.exp(t - m), axis=1, keepdims=True))
    out_ref[...] = t - m - ls


def _fin_call(a0, a1, y, dis, b2d):
    return pl.pallas_call(
        _fin_body,
        out_shape=jax.ShapeDtypeStruct((N, D_OUT), jnp.float32),
    )(a0, a1, y, dis, b2d)


@jax.jit
def kernel(x, edge_index, W, b):
    row = edge_index[0]
    col = edge_index[1]
    pad = E_PAD - E
    rowp = jnp.concatenate(
        [row, jnp.zeros((pad,), jnp.int32)]).reshape(NW, NCHUNK, CH)
    colp = jnp.concatenate(
        [col, jnp.full((pad,), N, jnp.int32)]).reshape(NW, NCHUNK, CH)

    ones_rows = jnp.ones((CH, D_OUT), jnp.float32)
    zeros_rows = jnp.zeros((RPT, D_OUT), jnp.float32)

    deg_parts = _deg_pass(colp, ones_rows, zeros_rows)
    d0 = deg_parts[0, :N]
    d1 = deg_parts[1, :N]

    y, dis = _xw_call(x, W, d0, d1)

    acc_parts = _edge_pass(y, rowp, colp, zeros_rows)
    a0 = acc_parts[0, :N]
    a1 = acc_parts[1, :N]

    return _fin_call(a0, a1, y, dis, b.reshape(1, D_OUT))


# speedup vs baseline: 46.2293x; 1.9331x over previous
import functools

import jax
import jax.numpy as jnp
from jax import lax
from jax.experimental import pallas as pl
from jax.experimental.pallas import tpu as pltpu
from jax.experimental.pallas import tpu_sc as plsc

N = 10000
E = 320000
D_IN = 128
D_OUT = 16

NC = 2
NS = 16
NW = NC * NS
CH = 128
NCHUNK = 80
E_PAD = NW * NCHUNK * CH
NP = 10240
RPT = NP // NS

NB = 8
NGRP = NCHUNK // NB
WIN = 20

_mesh = plsc.VectorSubcoreMesh(core_axis_name="c", subcore_axis_name="s")
_sc_params = pltpu.CompilerParams(use_tc_tiling_on_sc=False)


@functools.partial(
    pl.kernel,
    mesh=_mesh,
    out_type=jax.ShapeDtypeStruct((NC, NP, D_OUT), jnp.float32),
    scratch_types=[
        pltpu.VMEM((NCHUNK, CH), jnp.int32),
        pltpu.VMEM((CH, D_OUT), jnp.float32),
        pltpu.VMEM_SHARED((NP, D_OUT), jnp.float32),
        pltpu.SemaphoreType.DMA,
    ],
    compiler_params=_sc_params,
)
def _deg_pass(col_hbm, ones_hbm, zeros_hbm, deg_hbm, cidx_v, one_v, deg_sh, sem):
    c = lax.axis_index("c")
    s = lax.axis_index("s")
    wid = c * NS + s
    pltpu.sync_copy(ones_hbm, one_v)
    pltpu.sync_copy(col_hbm.at[wid], cidx_v)
    pltpu.sync_copy(zeros_hbm, deg_sh.at[pl.ds(s * RPT, RPT)])
    plsc.subcore_barrier()

    def fire(j):
        pltpu.async_copy(one_v, deg_sh.at[cidx_v.at[j]], sem, add=True)

    def wait_one():
        pltpu.make_async_copy(one_v, deg_sh.at[cidx_v.at[0]], sem).wait()

    def prol(j, carry):
        fire(j)
        return carry

    lax.fori_loop(0, WIN, prol, 0)

    def steady(j, carry):
        wait_one()
        fire(j + WIN)
        return carry

    lax.fori_loop(0, NCHUNK - WIN, steady, 0)

    def drain(j, carry):
        wait_one()
        return carry

    lax.fori_loop(0, WIN, drain, 0)
    plsc.subcore_barrier()
    pltpu.sync_copy(
        deg_sh.at[pl.ds(s * RPT, RPT)], deg_hbm.at[c, pl.ds(s * RPT, RPT)]
    )


@functools.partial(
    pl.kernel,
    mesh=_mesh,
    out_type=jax.ShapeDtypeStruct((NC, NP, D_OUT), jnp.float32),
    scratch_types=(
        [
            pltpu.VMEM((NCHUNK, CH), jnp.int32),
            pltpu.VMEM((NCHUNK, CH), jnp.int32),
            pltpu.VMEM((NB, CH, D_OUT), jnp.float32),
            pltpu.VMEM_SHARED((NP, D_OUT), jnp.float32),
        ]
        + [pltpu.SemaphoreType.DMA] * (2 * NB)
    ),
    compiler_params=_sc_params,
)
def _edge_pass(y_hbm, row_hbm, col_hbm, zeros_hbm, acc_hbm,
               ridx_v, cidx_v, rows_v, acc_sh, *sems):
    gsem = sems[:NB]
    ssem = sems[NB:]
    c = lax.axis_index("c")
    s = lax.axis_index("s")
    wid = c * NS + s
    pltpu.sync_copy(row_hbm.at[wid], ridx_v)
    pltpu.sync_copy(col_hbm.at[wid], cidx_v)
    pltpu.sync_copy(zeros_hbm, acc_sh.at[pl.ds(s * RPT, RPT)])
    plsc.subcore_barrier()

    for b in range(NB):
        pltpu.async_copy(y_hbm.at[ridx_v.at[b]], rows_v.at[b], gsem[b])

    def group(jo, carry):
        for b in range(NB):
            j = jo * NB + b
            pltpu.make_async_copy(
                y_hbm.at[ridx_v.at[j]], rows_v.at[b], gsem[b]).wait()
            pltpu.async_copy(
                rows_v.at[b], acc_sh.at[cidx_v.at[j]], ssem[b], add=True)
            pltpu.make_async_copy(
                rows_v.at[b], acc_sh.at[cidx_v.at[j]], ssem[b]).wait()
            pltpu.async_copy(
                y_hbm.at[ridx_v.at[j + NB]], rows_v.at[b], gsem[b])
        return carry

    lax.fori_loop(0, NGRP - 1, group, 0)

    for b in range(NB):
        j = (NGRP - 1) * NB + b
        pltpu.make_async_copy(
            y_hbm.at[ridx_v.at[j]], rows_v.at[b], gsem[b]).wait()
        pltpu.async_copy(
            rows_v.at[b], acc_sh.at[cidx_v.at[j]], ssem[b], add=True)
    for b in range(NB):
        j = (NGRP - 1) * NB + b
        pltpu.make_async_copy(
            rows_v.at[b], acc_sh.at[cidx_v.at[j]], ssem[b]).wait()

    plsc.subcore_barrier()
    pltpu.sync_copy(
        acc_sh.at[pl.ds(s * RPT, RPT)], acc_hbm.at[c, pl.ds(s * RPT, RPT)]
    )


def _xw_body(x_ref, w_ref, deg_ref, y_ref, dis_ref):
    deg = deg_ref[0, :N, :] + deg_ref[1, :N, :] + 1.0
    dis = lax.rsqrt(deg)
    xw = jnp.dot(x_ref[...], w_ref[...], preferred_element_type=jnp.float32)
    y_ref[...] = xw * dis
    dis_ref[...] = dis


def _xw_call(x, W, deg_parts):
    return pl.pallas_call(
        _xw_body,
        out_shape=[
            jax.ShapeDtypeStruct((N, D_OUT), jnp.float32),
            jax.ShapeDtypeStruct((N, D_OUT), jnp.float32),
        ],
    )(x, W, deg_parts)


def _fin_body(acc_ref, y_ref, dis_ref, b_ref, out_ref):
    t = (acc_ref[0, :N, :] + acc_ref[1, :N, :] + y_ref[...]) * dis_ref[...]
    t = t + b_ref[...]
    m = jnp.max(t, axis=1, keepdims=True)
    ls = jnp.log(jnp.sum(jnp.exp(t - m), axis=1, keepdims=True))
    out_ref[...] = t - m - ls


def _fin_call(acc_parts, y, dis, b2d):
    return pl.pallas_call(
        _fin_body,
        out_shape=jax.ShapeDtypeStruct((N, D_OUT), jnp.float32),
    )(acc_parts, y, dis, b2d)


@jax.jit
def kernel(x, edge_index, W, b):
    row = edge_index[0]
    col = edge_index[1]
    pad = E_PAD - E
    rowp = jnp.concatenate(
        [row, jnp.zeros((pad,), jnp.int32)]).reshape(NW, NCHUNK, CH)
    colp = jnp.concatenate(
        [col, jnp.full((pad,), N, jnp.int32)]).reshape(NW, NCHUNK, CH)

    ones_rows = jnp.ones((CH, D_OUT), jnp.float32)
    zeros_rows = jnp.zeros((RPT, D_OUT), jnp.float32)

    deg_parts = _deg_pass(colp, ones_rows, zeros_rows)
    y, dis = _xw_call(x, W, deg_parts)
    acc_parts = _edge_pass(y, rowp, colp, zeros_rows)
    return _fin_call(acc_parts, y, dis, b.reshape(1, D_OUT))
